# baseline (speedup 1.0000x reference)
import jax
import jax.numpy as jnp
from jax.experimental import pallas as pl
from jax.experimental.pallas import tpu as pltpu

B, S, D = 4, 4096, 1024
CR = 256
T = (B * S) // CR
NP = S // CR
NB = 8
AH = NB - 1


def _body(x_hbm, p_hbm, o_hbm, x_v, o_v, p_v, lsem, ssem, psem):
    def xload(i):
        return pltpu.make_async_copy(
            x_hbm.at[pl.ds(i * CR, CR), :], x_v.at[i % NB], lsem.at[i % NB])

    def pload(i):
        return pltpu.make_async_copy(
            p_hbm.at[pl.ds(i * CR, CR), :], p_v.at[pl.ds(i * CR, CR), :],
            psem.at[i])

    def store(i):
        return pltpu.make_async_copy(
            o_v.at[i % NB], o_hbm.at[pl.ds(i * CR, CR), :], ssem.at[i % NB])

    for i in range(AH):
        xload(i).start()
        pload(i).start()

    for i in range(T):
        if i + AH < T:
            xload(i + AH).start()
            if i + AH < NP:
                pload(i + AH).start()
        xload(i).wait()
        if i < NP:
            pload(i).wait()
        if i >= NB:
            store(i - NB).wait()
        pr = (i % NP) * CR
        o_v[i % NB] = x_v[i % NB] + p_v[pl.ds(pr, CR), :]
        store(i).start()

    for i in range(T - NB, T):
        store(i).wait()


def kernel(inputs, pos_embedding):
    x = jnp.reshape(inputs, (B * S, D))
    p = jnp.reshape(pos_embedding, (S, D))
    out = pl.pallas_call(
        _body,
        in_specs=[
            pl.BlockSpec(memory_space=pl.ANY),
            pl.BlockSpec(memory_space=pl.ANY),
        ],
        out_specs=pl.BlockSpec(memory_space=pl.ANY),
        out_shape=jax.ShapeDtypeStruct((B * S, D), jnp.float32),
        scratch_shapes=[
            pltpu.VMEM((NB, CR, D), jnp.float32),
            pltpu.VMEM((NB, CR, D), jnp.float32),
            pltpu.VMEM((S, D), jnp.float32),
            pltpu.SemaphoreType.DMA((NB,)),
            pltpu.SemaphoreType.DMA((NB,)),
            pltpu.SemaphoreType.DMA((NP,)),
        ],
    )(x, p)
    return jnp.reshape(out, (B, S, D))


# speedup vs baseline: 1.0048x; 1.0048x over previous
import jax
import jax.numpy as jnp
from jax.experimental import pallas as pl
from jax.experimental.pallas import tpu as pltpu

B, S, D = 4, 4096, 1024
CR = 512
T = (B * S) // CR
NP = S // CR
NB = 4
AH = NB - 1


def _body(x_hbm, p_hbm, o_hbm, x_v, o_v, p_v, lsem, ssem, psem):
    def xload(i):
        return pltpu.make_async_copy(
            x_hbm.at[pl.ds(i * CR, CR), :], x_v.at[i % NB], lsem.at[i % NB])

    def pload(i):
        return pltpu.make_async_copy(
            p_hbm.at[pl.ds(i * CR, CR), :], p_v.at[pl.ds(i * CR, CR), :],
            psem.at[i])

    def store(i):
        return pltpu.make_async_copy(
            o_v.at[i % NB], o_hbm.at[pl.ds(i * CR, CR), :], ssem.at[i % NB])

    for i in range(AH):
        xload(i).start()
        pload(i).start()

    for i in range(T):
        if i + AH < T:
            xload(i + AH).start()
            if i + AH < NP:
                pload(i + AH).start()
        xload(i).wait()
        if i < NP:
            pload(i).wait()
        if i >= NB:
            store(i - NB).wait()
        pr = (i % NP) * CR
        o_v[i % NB] = x_v[i % NB] + p_v[pl.ds(pr, CR), :]
        store(i).start()

    for i in range(T - NB, T):
        store(i).wait()


def kernel(inputs, pos_embedding):
    x = jnp.reshape(inputs, (B * S, D))
    p = jnp.reshape(pos_embedding, (S, D))
    out = pl.pallas_call(
        _body,
        in_specs=[
            pl.BlockSpec(memory_space=pl.ANY),
            pl.BlockSpec(memory_space=pl.ANY),
        ],
        out_specs=pl.BlockSpec(memory_space=pl.ANY),
        out_shape=jax.ShapeDtypeStruct((B * S, D), jnp.float32),
        scratch_shapes=[
            pltpu.VMEM((NB, CR, D), jnp.float32),
            pltpu.VMEM((NB, CR, D), jnp.float32),
            pltpu.VMEM((S, D), jnp.float32),
            pltpu.SemaphoreType.DMA((NB,)),
            pltpu.SemaphoreType.DMA((NB,)),
            pltpu.SemaphoreType.DMA((NP,)),
        ],
    )(x, p)
    return jnp.reshape(out, (B, S, D))
